# baseline (speedup 1.0000x reference)
import jax
import jax.numpy as jnp
from jax import lax
from jax.experimental import pallas as pl
from jax.experimental.pallas import tpu as pltpu
from jax.experimental.pallas import tpu_sc as plsc

NCS = 1
NS = 16
LANES = 16
K_EDGES = 64


def _round_up(a, b):
  return (a + b - 1) // b * b


def _tc_in_body(x_ref, w_ref, asr_ref, adr_ref, h_ref, asad_ref):
  h = jnp.dot(x_ref[...], w_ref[...], preferred_element_type=jnp.float32)
  h_ref[...] = h
  asad_ref[0, :] = jnp.sum(h * asr_ref[...], axis=1)
  asad_ref[1, :] = jnp.sum(h * adr_ref[...], axis=1)


def _tc_mid_body(acc_ref, den_ref, b_ref, w_ref, asr_ref, adr_ref,
                 h_ref, asad_ref):
  d = jnp.sum(den_ref[...], axis=0)
  hprev = acc_ref[0] / (d[:, None] + 1e-16) + b_ref[...]
  hprev = jnp.maximum(hprev, 0.0)
  h = jnp.dot(hprev, w_ref[...], preferred_element_type=jnp.float32)
  h_ref[...] = h
  asad_ref[0, :] = jnp.sum(h * asr_ref[...], axis=1)
  asad_ref[1, :] = jnp.sum(h * adr_ref[...], axis=1)


def _tc_head_body(acc_ref, den_ref, b_ref, wo_ref, bo_ref, o_ref):
  d = jnp.sum(den_ref[...], axis=0)
  hprev = acc_ref[0] / (d[:, None] + 1e-16) + b_ref[...]
  hprev = jnp.maximum(hprev, 0.0)
  logits = jnp.dot(hprev, wo_ref[...], preferred_element_type=jnp.float32)
  logits = logits + bo_ref[...]
  m = jnp.max(logits, axis=1, keepdims=True)
  p = jnp.exp(logits - m)
  o_ref[...] = p / jnp.sum(p, axis=1, keepdims=True)


def _sc_edge_call(n_pad, e_pad, d, h, srcr, dstr, asad):
  stripe = n_pad // NS
  cpw = e_pad // (NS * K_EDGES)
  n16 = n_pad // LANES

  def body(h_hbm, srcr_hbm, dstr_hbm, asad_hbm, acc_hbm, den_hbm,
           asrc_v, adst_v, den_v, sidx_v, didx_v, ex_v, rows_v, acc_s,
           sem_g, sem_s):
    s = lax.axis_index("s")

    pltpu.sync_copy(asad_hbm.at[0], asrc_v)
    pltpu.sync_copy(asad_hbm.at[1], adst_v)

    zero16 = jnp.zeros((LANES,), jnp.float32)

    def zrow(i, carry):
      for r8 in range(d // LANES):
        rows_v[0, i, pl.ds(r8 * LANES, LANES)] = zero16
      return carry
    lax.fori_loop(0, K_EDGES, zrow, 0)

    def zden(i, carry):
      den_v[pl.ds(i * LANES, LANES)] = zero16
      return carry
    lax.fori_loop(0, n16, zden, 0)

    off = 0
    while off < stripe:
      step = min(K_EDGES, stripe - off)
      pltpu.sync_copy(rows_v.at[0].at[pl.ds(0, step)],
                      acc_s.at[pl.ds(s * stripe + off, step)])
      off += step
    plsc.subcore_barrier()

    row0 = s * cpw

    pltpu.sync_copy(srcr_hbm.at[row0], sidx_v.at[0])
    pltpu.sync_copy(dstr_hbm.at[row0], didx_v.at[0])
    pltpu.async_copy(h_hbm.at[sidx_v.at[0]], rows_v.at[0], sem_g.at[0])

    def chunk(ci, carry):
      p = lax.rem(ci, 2)
      q = 1 - p
      rbase = row0 + ci

      @pl.when(ci >= 1)
      def _wait_prev_scatter():
        pltpu.make_async_copy(rows_v.at[q], acc_s.at[didx_v.at[q]],
                              sem_s.at[q]).wait()

      @pl.when(ci + 1 < cpw)
      def _prefetch_next():
        pltpu.sync_copy(srcr_hbm.at[rbase + 1], sidx_v.at[q])
        pltpu.sync_copy(dstr_hbm.at[rbase + 1], didx_v.at[q])
        pltpu.async_copy(h_hbm.at[sidx_v.at[q]], rows_v.at[q],
                         sem_g.at[q])

      for cc in range(K_EDGES // LANES):
        si = sidx_v[p, pl.ds(cc * LANES, LANES)]
        di = didx_v[p, pl.ds(cc * LANES, LANES)]
        t = (plsc.load_gather(asrc_v, [si])
             + plsc.load_gather(adst_v, [di]))
        ex = jnp.exp(jnp.maximum(t, t * 0.2))
        ex_v[pl.ds(cc * LANES, LANES)] = ex
        plsc.addupdate_scatter(den_v, [di], ex)

      pltpu.make_async_copy(h_hbm.at[sidx_v.at[p]], rows_v.at[p],
                            sem_g.at[p]).wait()

      def scale(g, carry2):
        exv = ex_v[pl.ds(g * LANES, LANES)]
        for l in range(LANES):
          xv = exv[l]
          row = g * LANES + l
          for r8 in range(d // LANES):
            sl = pl.ds(r8 * LANES, LANES)
            rows_v[p, row, sl] = rows_v[p, row, sl] * xv
        return carry2
      lax.fori_loop(0, K_EDGES // LANES, scale, 0)

      pltpu.async_copy(rows_v.at[p], acc_s.at[didx_v.at[p]], sem_s.at[p],
                       add=True)
      return carry

    lax.fori_loop(0, cpw, chunk, 0)
    lastp = (cpw - 1) % 2
    pltpu.make_async_copy(rows_v.at[lastp], acc_s.at[didx_v.at[lastp]],
                          sem_s.at[lastp]).wait()
    plsc.subcore_barrier()

    pltpu.sync_copy(den_v, den_hbm.at[s])
    pltpu.sync_copy(acc_s.at[pl.ds(s * stripe, stripe)],
                    acc_hbm.at[0].at[pl.ds(s * stripe, stripe)])

  mesh = plsc.VectorSubcoreMesh(core_axis_name="c", subcore_axis_name="s",
                                num_cores=NCS, num_subcores=NS)
  fn = pl.kernel(
      body,
      out_type=[jax.ShapeDtypeStruct((NCS, n_pad, d), jnp.float32),
                jax.ShapeDtypeStruct((NS, n_pad), jnp.float32)],
      mesh=mesh,
      scratch_types=[
          pltpu.VMEM((n_pad,), jnp.float32),
          pltpu.VMEM((n_pad,), jnp.float32),
          pltpu.VMEM((n_pad,), jnp.float32),
          pltpu.VMEM((2, K_EDGES), jnp.int32),
          pltpu.VMEM((2, K_EDGES), jnp.int32),
          pltpu.VMEM((K_EDGES,), jnp.float32),
          pltpu.VMEM((2, K_EDGES, d), jnp.float32),
          pltpu.VMEM_SHARED((n_pad, d), jnp.float32),
          pltpu.SemaphoreType.DMA((2,)),
          pltpu.SemaphoreType.DMA((2,)),
      ],
      compiler_params=pltpu.CompilerParams(needs_layout_passes=False),
  )
  return fn(h, srcr, dstr, asad)


def kernel(x, edge_index, W1, a_src1, a_dst1, b1, W2, a_src2, a_dst2, b2,
           Wo, bo):
  n, d_in = x.shape
  d_h = W1.shape[1]
  n_classes = Wo.shape[1]
  e = edge_index.shape[1]
  n_pad = _round_up(n + 1, 128)
  e_pad = _round_up(e + n, NS * K_EDGES)

  ei = edge_index.astype(jnp.int32)
  loop = jnp.arange(n, dtype=jnp.int32)
  pad_e = e_pad - e - n
  src = jnp.concatenate(
      [ei[0], loop, jnp.zeros((pad_e,), jnp.int32)]).reshape(-1, K_EDGES)
  dst = jnp.concatenate(
      [ei[1], loop, jnp.full((pad_e,), n, jnp.int32)]).reshape(-1, K_EDGES)

  x_pad = jnp.pad(x, ((0, n_pad - n), (0, 0)))

  vec = lambda a: a.reshape(1, -1)

  h1, asad1 = pl.pallas_call(
      _tc_in_body,
      out_shape=[jax.ShapeDtypeStruct((n_pad, d_h), jnp.float32),
                 jax.ShapeDtypeStruct((2, n_pad), jnp.float32)],
  )(x_pad, W1, vec(a_src1), vec(a_dst1))

  acc1, den1 = _sc_edge_call(n_pad, e_pad, d_h, h1, src, dst, asad1)

  h2, asad2 = pl.pallas_call(
      _tc_mid_body,
      out_shape=[jax.ShapeDtypeStruct((n_pad, d_h), jnp.float32),
                 jax.ShapeDtypeStruct((2, n_pad), jnp.float32)],
  )(acc1, den1, vec(b1), W2, vec(a_src2), vec(a_dst2))

  acc2, den2 = _sc_edge_call(n_pad, e_pad, d_h, h2, src, dst, asad2)

  out = pl.pallas_call(
      _tc_head_body,
      out_shape=jax.ShapeDtypeStruct((n_pad, n_classes), jnp.float32),
  )(acc2, den2, vec(b2), Wo, vec(bo))

  return out[:n]


# speedup vs baseline: 1.8364x; 1.8364x over previous
import jax
import jax.numpy as jnp
from jax import lax
from jax.experimental import pallas as pl
from jax.experimental.pallas import tpu as pltpu
from jax.experimental.pallas import tpu_sc as plsc

NCS = 1
NS = 16
LANES = 16
K_EDGES = 128


def _round_up(a, b):
  return (a + b - 1) // b * b


def _tc_in_body(x_ref, w_ref, asr_ref, adr_ref, h_ref, asad_ref):
  h = jnp.dot(x_ref[...], w_ref[...], preferred_element_type=jnp.float32)
  h_ref[...] = h
  asad_ref[0, :] = jnp.sum(h * asr_ref[...], axis=1)
  asad_ref[1, :] = jnp.sum(h * adr_ref[...], axis=1)


def _tc_mid_body(acc_ref, den_ref, b_ref, w_ref, asr_ref, adr_ref,
                 h_ref, asad_ref):
  d = jnp.sum(den_ref[...], axis=0)
  hprev = acc_ref[0] / (d[:, None] + 1e-16) + b_ref[...]
  hprev = jnp.maximum(hprev, 0.0)
  h = jnp.dot(hprev, w_ref[...], preferred_element_type=jnp.float32)
  h_ref[...] = h
  asad_ref[0, :] = jnp.sum(h * asr_ref[...], axis=1)
  asad_ref[1, :] = jnp.sum(h * adr_ref[...], axis=1)


def _tc_head_body(acc_ref, den_ref, b_ref, wo_ref, bo_ref, o_ref):
  d = jnp.sum(den_ref[...], axis=0)
  hprev = acc_ref[0] / (d[:, None] + 1e-16) + b_ref[...]
  hprev = jnp.maximum(hprev, 0.0)
  logits = jnp.dot(hprev, wo_ref[...], preferred_element_type=jnp.float32)
  logits = logits + bo_ref[...]
  m = jnp.max(logits, axis=1, keepdims=True)
  p = jnp.exp(logits - m)
  o_ref[...] = p / jnp.sum(p, axis=1, keepdims=True)


def _sc_edge_call(n_pad, e_pad, d, h, srcr, dstr, asad):
  stripe = n_pad // NS
  cpw = e_pad // (NS * K_EDGES)
  n16 = n_pad // LANES

  def body(h_hbm, srcr_hbm, dstr_hbm, asad_hbm, acc_hbm, den_hbm,
           asrc_v, adst_v, den_v, sidx_v, didx_v, ex_v, rows_v, acc_s,
           sem_g, sem_s):
    s = lax.axis_index("s")

    pltpu.sync_copy(asad_hbm.at[0], asrc_v)
    pltpu.sync_copy(asad_hbm.at[1], adst_v)

    zero16 = jnp.zeros((LANES,), jnp.float32)

    def zrow(i, carry):
      for r8 in range(d // LANES):
        rows_v[i, pl.ds(r8 * LANES, LANES)] = zero16
      return carry
    lax.fori_loop(0, K_EDGES, zrow, 0)

    def zden(i, carry):
      den_v[pl.ds(i * LANES, LANES)] = zero16
      return carry
    lax.fori_loop(0, n16, zden, 0)

    off = 0
    while off < stripe:
      step = min(K_EDGES, stripe - off)
      pltpu.sync_copy(rows_v.at[pl.ds(0, step)],
                      acc_s.at[pl.ds(s * stripe + off, step)])
      off += step
    plsc.subcore_barrier()

    row0 = s * cpw

    def chunk(ci, carry):
      rbase = row0 + ci
      pltpu.sync_copy(srcr_hbm.at[pl.ds(rbase, 1)], sidx_v)
      pltpu.sync_copy(dstr_hbm.at[pl.ds(rbase, 1)], didx_v)
      gcp = pltpu.async_copy(h_hbm.at[sidx_v.at[0]], rows_v, sem_g)
      for cc in range(K_EDGES // LANES):
        si = sidx_v[0, pl.ds(cc * LANES, LANES)]
        di = didx_v[0, pl.ds(cc * LANES, LANES)]
        t = (plsc.load_gather(asrc_v, [si])
             + plsc.load_gather(adst_v, [di]))
        ex = jnp.exp(jnp.maximum(t, t * 0.2))
        ex_v[pl.ds(cc * LANES, LANES)] = ex
        plsc.addupdate_scatter(den_v, [di], ex)
      gcp.wait()

      def scale(g, carry2):
        exv = ex_v[pl.ds(g * LANES, LANES)]
        for l in range(LANES):
          xv = exv[l]
          row = g * LANES + l
          for r8 in range(d // LANES):
            sl = pl.ds(r8 * LANES, LANES)
            rows_v[row, sl] = rows_v[row, sl] * xv
        return carry2
      lax.fori_loop(0, K_EDGES // LANES, scale, 0)

      pltpu.async_copy(rows_v, acc_s.at[didx_v.at[0]], sem_s,
                       add=True).wait()
      return carry

    lax.fori_loop(0, cpw, chunk, 0)
    plsc.subcore_barrier()

    pltpu.sync_copy(den_v, den_hbm.at[s])
    pltpu.sync_copy(acc_s.at[pl.ds(s * stripe, stripe)],
                    acc_hbm.at[0].at[pl.ds(s * stripe, stripe)])

  mesh = plsc.VectorSubcoreMesh(core_axis_name="c", subcore_axis_name="s",
                                num_cores=NCS, num_subcores=NS)
  fn = pl.kernel(
      body,
      out_type=[jax.ShapeDtypeStruct((NCS, n_pad, d), jnp.float32),
                jax.ShapeDtypeStruct((NS, n_pad), jnp.float32)],
      mesh=mesh,
      scratch_types=[
          pltpu.VMEM((n_pad,), jnp.float32),
          pltpu.VMEM((n_pad,), jnp.float32),
          pltpu.VMEM((n_pad,), jnp.float32),
          pltpu.VMEM((1, K_EDGES), jnp.int32),
          pltpu.VMEM((1, K_EDGES), jnp.int32),
          pltpu.VMEM((K_EDGES,), jnp.float32),
          pltpu.VMEM((K_EDGES, d), jnp.float32),
          pltpu.VMEM_SHARED((n_pad, d), jnp.float32),
          pltpu.SemaphoreType.DMA,
          pltpu.SemaphoreType.DMA,
      ],
      compiler_params=pltpu.CompilerParams(needs_layout_passes=False),
  )
  return fn(h, srcr, dstr, asad)


def kernel(x, edge_index, W1, a_src1, a_dst1, b1, W2, a_src2, a_dst2, b2,
           Wo, bo):
  n, d_in = x.shape
  d_h = W1.shape[1]
  n_classes = Wo.shape[1]
  e = edge_index.shape[1]
  n_pad = _round_up(n + 1, 128)
  e_pad = _round_up(e + n, NS * K_EDGES)

  ei = edge_index.astype(jnp.int32)
  loop = jnp.arange(n, dtype=jnp.int32)
  pad_e = e_pad - e - n
  src = jnp.concatenate(
      [ei[0], loop, jnp.zeros((pad_e,), jnp.int32)]).reshape(-1, K_EDGES)
  dst = jnp.concatenate(
      [ei[1], loop, jnp.full((pad_e,), n, jnp.int32)]).reshape(-1, K_EDGES)

  x_pad = jnp.pad(x, ((0, n_pad - n), (0, 0)))

  vec = lambda a: a.reshape(1, -1)

  h1, asad1 = pl.pallas_call(
      _tc_in_body,
      out_shape=[jax.ShapeDtypeStruct((n_pad, d_h), jnp.float32),
                 jax.ShapeDtypeStruct((2, n_pad), jnp.float32)],
  )(x_pad, W1, vec(a_src1), vec(a_dst1))

  acc1, den1 = _sc_edge_call(n_pad, e_pad, d_h, h1, src, dst, asad1)

  h2, asad2 = pl.pallas_call(
      _tc_mid_body,
      out_shape=[jax.ShapeDtypeStruct((n_pad, d_h), jnp.float32),
                 jax.ShapeDtypeStruct((2, n_pad), jnp.float32)],
  )(acc1, den1, vec(b1), W2, vec(a_src2), vec(a_dst2))

  acc2, den2 = _sc_edge_call(n_pad, e_pad, d_h, h2, src, dst, asad2)

  out = pl.pallas_call(
      _tc_head_body,
      out_shape=jax.ShapeDtypeStruct((n_pad, n_classes), jnp.float32),
  )(acc2, den2, vec(b2), Wo, vec(bo))

  return out[:n]


# speedup vs baseline: 2.4089x; 1.3117x over previous
import jax
import jax.numpy as jnp
from jax import lax
from jax.experimental import pallas as pl
from jax.experimental.pallas import tpu as pltpu
from jax.experimental.pallas import tpu_sc as plsc

NCS = 1
NS = 16
LANES = 16
K_EDGES = 64


def _round_up(a, b):
  return (a + b - 1) // b * b


def _tc_in_body(x_ref, w_ref, asr_ref, adr_ref, h_ref, asad_ref):
  h = jnp.dot(x_ref[...], w_ref[...], preferred_element_type=jnp.float32)
  h_ref[...] = h
  asad_ref[0, :] = jnp.sum(h * asr_ref[...], axis=1)
  asad_ref[1, :] = jnp.sum(h * adr_ref[...], axis=1)


def _tc_mid_body(acc_ref, den_ref, b_ref, w_ref, asr_ref, adr_ref,
                 h_ref, asad_ref):
  d = jnp.sum(den_ref[...], axis=0)
  hprev = acc_ref[0] / (d[:, None] + 1e-16) + b_ref[...]
  hprev = jnp.maximum(hprev, 0.0)
  h = jnp.dot(hprev, w_ref[...], preferred_element_type=jnp.float32)
  h_ref[...] = h
  asad_ref[0, :] = jnp.sum(h * asr_ref[...], axis=1)
  asad_ref[1, :] = jnp.sum(h * adr_ref[...], axis=1)


def _tc_head_body(acc_ref, den_ref, b_ref, wo_ref, bo_ref, o_ref):
  d = jnp.sum(den_ref[...], axis=0)
  hprev = acc_ref[0] / (d[:, None] + 1e-16) + b_ref[...]
  hprev = jnp.maximum(hprev, 0.0)
  logits = jnp.dot(hprev, wo_ref[...], preferred_element_type=jnp.float32)
  logits = logits + bo_ref[...]
  m = jnp.max(logits, axis=1, keepdims=True)
  p = jnp.exp(logits - m)
  o_ref[...] = p / jnp.sum(p, axis=1, keepdims=True)


def _sc_edge_call(n_pad, e_pad, d, h, sd, asad):
  stripe = n_pad // NS
  cpw = e_pad // (NS * K_EDGES)
  assert cpw % 2 == 0
  n16 = n_pad // LANES

  def body(h_hbm, sd_hbm, asad_hbm, acc_hbm, den_hbm,
           asrc_v, adst_v, den_v, sd_a, sd_b, ex_v, rows_a, rows_b,
           acc_s, sem_ga, sem_gb, sem_sa, sem_sb):
    s = lax.axis_index("s")

    pltpu.sync_copy(asad_hbm.at[0], asrc_v)
    pltpu.sync_copy(asad_hbm.at[1], adst_v)

    zero16 = jnp.zeros((LANES,), jnp.float32)

    def zrow(i, carry):
      for r8 in range(d // LANES):
        rows_a[i, pl.ds(r8 * LANES, LANES)] = zero16
      return carry
    lax.fori_loop(0, K_EDGES, zrow, 0)

    def zden(i, carry):
      den_v[pl.ds(i * LANES, LANES)] = zero16
      return carry
    lax.fori_loop(0, n16, zden, 0)

    off = 0
    while off < stripe:
      step = min(K_EDGES, stripe - off)
      pltpu.sync_copy(rows_a.at[pl.ds(0, step)],
                      acc_s.at[pl.ds(s * stripe + off, step)])
      off += step
    plsc.subcore_barrier()

    row0 = s * cpw

    def fire_gather(row, sd_v, rows_b_, g_sem):
      pltpu.sync_copy(sd_hbm.at[row], sd_v)
      pltpu.async_copy(h_hbm.at[sd_v.at[0]], rows_b_, g_sem)

    def do_chunk(sd_v, rows_b_, g_sem, s_sem):
      for cc in range(K_EDGES // LANES):
        si = sd_v[0, pl.ds(cc * LANES, LANES)]
        di = sd_v[1, pl.ds(cc * LANES, LANES)]
        t = (plsc.load_gather(asrc_v, [si])
             + plsc.load_gather(adst_v, [di]))
        ex = jnp.exp(jnp.maximum(t, t * 0.2))
        ex_v[pl.ds(cc * LANES, LANES)] = ex
        plsc.addupdate_scatter(den_v, [di], ex)
      pltpu.make_async_copy(h_hbm.at[sd_v.at[0]], rows_b_, g_sem).wait()

      def scale(g, carry2):
        exv = ex_v[pl.ds(g * LANES, LANES)]
        for l in range(LANES):
          xv = exv[l]
          row = g * LANES + l
          for r8 in range(d // LANES):
            sl = pl.ds(r8 * LANES, LANES)
            rows_b_[row, sl] = rows_b_[row, sl] * xv
        return carry2
      lax.fori_loop(0, K_EDGES // LANES, scale, 0)

      pltpu.async_copy(rows_b_, acc_s.at[sd_v.at[1]], s_sem, add=True)

    fire_gather(row0, sd_a, rows_a, sem_ga)
    fire_gather(row0 + 1, sd_b, rows_b, sem_gb)

    def pair(i2, carry):
      c0 = row0 + 2 * i2
      do_chunk(sd_a, rows_a, sem_ga, sem_sa)
      do_chunk(sd_b, rows_b, sem_gb, sem_sb)
      pltpu.make_async_copy(rows_a, acc_s.at[sd_a.at[1]], sem_sa).wait()
      fire_gather(c0 + 2, sd_a, rows_a, sem_ga)
      pltpu.make_async_copy(rows_b, acc_s.at[sd_b.at[1]], sem_sb).wait()
      fire_gather(c0 + 3, sd_b, rows_b, sem_gb)
      return carry
    lax.fori_loop(0, cpw // 2, pair, 0)

    pltpu.make_async_copy(h_hbm.at[sd_a.at[0]], rows_a, sem_ga).wait()
    pltpu.make_async_copy(h_hbm.at[sd_b.at[0]], rows_b, sem_gb).wait()
    plsc.subcore_barrier()

    pltpu.sync_copy(den_v, den_hbm.at[s])
    pltpu.sync_copy(acc_s.at[pl.ds(s * stripe, stripe)],
                    acc_hbm.at[0].at[pl.ds(s * stripe, stripe)])

  mesh = plsc.VectorSubcoreMesh(core_axis_name="c", subcore_axis_name="s",
                                num_cores=NCS, num_subcores=NS)
  fn = pl.kernel(
      body,
      out_type=[jax.ShapeDtypeStruct((NCS, n_pad, d), jnp.float32),
                jax.ShapeDtypeStruct((NS, n_pad), jnp.float32)],
      mesh=mesh,
      scratch_types=[
          pltpu.VMEM((n_pad,), jnp.float32),
          pltpu.VMEM((n_pad,), jnp.float32),
          pltpu.VMEM((n_pad,), jnp.float32),
          pltpu.VMEM((2, K_EDGES), jnp.int32),
          pltpu.VMEM((2, K_EDGES), jnp.int32),
          pltpu.VMEM((K_EDGES,), jnp.float32),
          pltpu.VMEM((K_EDGES, d), jnp.float32),
          pltpu.VMEM((K_EDGES, d), jnp.float32),
          pltpu.VMEM_SHARED((n_pad, d), jnp.float32),
          pltpu.SemaphoreType.DMA,
          pltpu.SemaphoreType.DMA,
          pltpu.SemaphoreType.DMA,
          pltpu.SemaphoreType.DMA,
      ],
      compiler_params=pltpu.CompilerParams(needs_layout_passes=False),
  )
  return fn(h, sd, asad)


def kernel(x, edge_index, W1, a_src1, a_dst1, b1, W2, a_src2, a_dst2, b2,
           Wo, bo):
  n, d_in = x.shape
  d_h = W1.shape[1]
  n_classes = Wo.shape[1]
  e = edge_index.shape[1]
  n_pad = _round_up(n + 1, 128)
  e_pad = _round_up(e + n, NS * K_EDGES * 2)
  e_tot = e_pad + 2 * K_EDGES

  ei = edge_index.astype(jnp.int32)
  loop = jnp.arange(n, dtype=jnp.int32)
  pad_e = e_tot - e - n
  srcf = jnp.concatenate(
      [ei[0], loop, jnp.zeros((pad_e,), jnp.int32)]).reshape(-1, K_EDGES)
  dstf = jnp.concatenate(
      [ei[1], loop, jnp.full((pad_e,), n, jnp.int32)]).reshape(-1, K_EDGES)
  sd = jnp.stack([srcf, dstf], axis=1)

  x_pad = jnp.pad(x, ((0, n_pad - n), (0, 0)))

  vec = lambda a: a.reshape(1, -1)

  h1, asad1 = pl.pallas_call(
      _tc_in_body,
      out_shape=[jax.ShapeDtypeStruct((n_pad, d_h), jnp.float32),
                 jax.ShapeDtypeStruct((2, n_pad), jnp.float32)],
  )(x_pad, W1, vec(a_src1), vec(a_dst1))

  acc1, den1 = _sc_edge_call(n_pad, e_pad, d_h, h1, sd, asad1)

  h2, asad2 = pl.pallas_call(
      _tc_mid_body,
      out_shape=[jax.ShapeDtypeStruct((n_pad, d_h), jnp.float32),
                 jax.ShapeDtypeStruct((2, n_pad), jnp.float32)],
  )(acc1, den1, vec(b1), W2, vec(a_src2), vec(a_dst2))

  acc2, den2 = _sc_edge_call(n_pad, e_pad, d_h, h2, sd, asad2)

  out = pl.pallas_call(
      _tc_head_body,
      out_shape=jax.ShapeDtypeStruct((n_pad, n_classes), jnp.float32),
  )(acc2, den2, vec(b2), Wo, vec(bo))

  return out[:n]


# speedup vs baseline: 3.7526x; 1.5578x over previous
import jax
import jax.numpy as jnp
from jax import lax
from jax.experimental import pallas as pl
from jax.experimental.pallas import tpu as pltpu
from jax.experimental.pallas import tpu_sc as plsc

NCS = 2
NS = 16
LANES = 16
K_EDGES = 64


def _round_up(a, b):
  return (a + b - 1) // b * b


def _tc_in_body(x_ref, w_ref, asr_ref, adr_ref, h_ref, asad_ref):
  h = jnp.dot(x_ref[...], w_ref[...], preferred_element_type=jnp.float32)
  h_ref[...] = h
  asad_ref[0, :] = jnp.sum(h * asr_ref[...], axis=1)
  asad_ref[1, :] = jnp.sum(h * adr_ref[...], axis=1)


def _tc_mid_body(acc_ref, den_ref, b_ref, w_ref, asr_ref, adr_ref,
                 h_ref, asad_ref):
  d = jnp.sum(den_ref[...], axis=0)
  agg = acc_ref[0] + acc_ref[1]
  hprev = agg / (d[:, None] + 1e-16) + b_ref[...]
  hprev = jnp.maximum(hprev, 0.0)
  h = jnp.dot(hprev, w_ref[...], preferred_element_type=jnp.float32)
  h_ref[...] = h
  asad_ref[0, :] = jnp.sum(h * asr_ref[...], axis=1)
  asad_ref[1, :] = jnp.sum(h * adr_ref[...], axis=1)


def _tc_head_body(acc_ref, den_ref, b_ref, wo_ref, bo_ref, o_ref):
  d = jnp.sum(den_ref[...], axis=0)
  agg = acc_ref[0] + acc_ref[1]
  hprev = agg / (d[:, None] + 1e-16) + b_ref[...]
  hprev = jnp.maximum(hprev, 0.0)
  logits = jnp.dot(hprev, wo_ref[...], preferred_element_type=jnp.float32)
  logits = logits + bo_ref[...]
  m = jnp.max(logits, axis=1, keepdims=True)
  p = jnp.exp(logits - m)
  o_ref[...] = p / jnp.sum(p, axis=1, keepdims=True)


def _sc_edge_call(n_pad, e_pad, d, h, sd, asad):
  stripe = n_pad // NS
  cpw = e_pad // (NCS * NS * K_EDGES)
  assert cpw % 2 == 0
  n16 = n_pad // LANES

  def body(h_hbm, sd_hbm, asad_hbm, acc_hbm, den_hbm,
           asrc_v, adst_v, den_v, sd_a, sd_b, ex_v, rows_a, rows_b,
           acc_s, sem_ga, sem_gb, sem_sa, sem_sb):
    c = lax.axis_index("c")
    s = lax.axis_index("s")
    wid = c * NS + s

    pltpu.sync_copy(asad_hbm.at[0], asrc_v)
    pltpu.sync_copy(asad_hbm.at[1], adst_v)

    zero16 = jnp.zeros((LANES,), jnp.float32)

    def zrow(i, carry):
      for r8 in range(d // LANES):
        rows_a[i, pl.ds(r8 * LANES, LANES)] = zero16
      return carry
    lax.fori_loop(0, K_EDGES, zrow, 0)

    def zden(i, carry):
      den_v[pl.ds(i * LANES, LANES)] = zero16
      return carry
    lax.fori_loop(0, n16, zden, 0)

    off = 0
    while off < stripe:
      step = min(K_EDGES, stripe - off)
      pltpu.sync_copy(rows_a.at[pl.ds(0, step)],
                      acc_s.at[pl.ds(s * stripe + off, step)])
      off += step
    plsc.subcore_barrier()

    row0 = wid * cpw

    def fire_gather(row, sd_v, rows_b_, g_sem):
      pltpu.sync_copy(sd_hbm.at[row], sd_v)
      pltpu.async_copy(h_hbm.at[sd_v.at[0]], rows_b_, g_sem)

    def do_chunk(sd_v, rows_b_, g_sem, s_sem):
      for cc in range(K_EDGES // LANES):
        si = sd_v[0, pl.ds(cc * LANES, LANES)]
        di = sd_v[1, pl.ds(cc * LANES, LANES)]
        t = (plsc.load_gather(asrc_v, [si])
             + plsc.load_gather(adst_v, [di]))
        ex = jnp.exp(jnp.maximum(t, t * 0.2))
        ex_v[pl.ds(cc * LANES, LANES)] = ex
        plsc.addupdate_scatter(den_v, [di], ex)
      pltpu.make_async_copy(h_hbm.at[sd_v.at[0]], rows_b_, g_sem).wait()

      def scale(g, carry2):
        exv = ex_v[pl.ds(g * LANES, LANES)]
        for l in range(LANES):
          xv = exv[l]
          row = g * LANES + l
          for r8 in range(d // LANES):
            sl = pl.ds(r8 * LANES, LANES)
            rows_b_[row, sl] = rows_b_[row, sl] * xv
        return carry2
      lax.fori_loop(0, K_EDGES // LANES, scale, 0)

      pltpu.async_copy(rows_b_, acc_s.at[sd_v.at[1]], s_sem, add=True)

    fire_gather(row0, sd_a, rows_a, sem_ga)
    fire_gather(row0 + 1, sd_b, rows_b, sem_gb)

    def pair(i2, carry):
      c0 = row0 + 2 * i2
      do_chunk(sd_a, rows_a, sem_ga, sem_sa)
      do_chunk(sd_b, rows_b, sem_gb, sem_sb)
      pltpu.make_async_copy(rows_a, acc_s.at[sd_a.at[1]], sem_sa).wait()
      fire_gather(c0 + 2, sd_a, rows_a, sem_ga)
      pltpu.make_async_copy(rows_b, acc_s.at[sd_b.at[1]], sem_sb).wait()
      fire_gather(c0 + 3, sd_b, rows_b, sem_gb)
      return carry
    lax.fori_loop(0, cpw // 2, pair, 0)

    pltpu.make_async_copy(h_hbm.at[sd_a.at[0]], rows_a, sem_ga).wait()
    pltpu.make_async_copy(h_hbm.at[sd_b.at[0]], rows_b, sem_gb).wait()
    plsc.subcore_barrier()

    pltpu.sync_copy(den_v, den_hbm.at[wid])
    pltpu.sync_copy(acc_s.at[pl.ds(s * stripe, stripe)],
                    acc_hbm.at[c].at[pl.ds(s * stripe, stripe)])

  mesh = plsc.VectorSubcoreMesh(core_axis_name="c", subcore_axis_name="s",
                                num_cores=NCS, num_subcores=NS)
  fn = pl.kernel(
      body,
      out_type=[jax.ShapeDtypeStruct((NCS, n_pad, d), jnp.float32),
                jax.ShapeDtypeStruct((NCS * NS, n_pad), jnp.float32)],
      mesh=mesh,
      scratch_types=[
          pltpu.VMEM((n_pad,), jnp.float32),
          pltpu.VMEM((n_pad,), jnp.float32),
          pltpu.VMEM((n_pad,), jnp.float32),
          pltpu.VMEM((2, K_EDGES), jnp.int32),
          pltpu.VMEM((2, K_EDGES), jnp.int32),
          pltpu.VMEM((K_EDGES,), jnp.float32),
          pltpu.VMEM((K_EDGES, d), jnp.float32),
          pltpu.VMEM((K_EDGES, d), jnp.float32),
          pltpu.VMEM_SHARED((n_pad, d), jnp.float32),
          pltpu.SemaphoreType.DMA,
          pltpu.SemaphoreType.DMA,
          pltpu.SemaphoreType.DMA,
          pltpu.SemaphoreType.DMA,
      ],
      compiler_params=pltpu.CompilerParams(needs_layout_passes=False),
  )
  return fn(h, sd, asad)


def kernel(x, edge_index, W1, a_src1, a_dst1, b1, W2, a_src2, a_dst2, b2,
           Wo, bo):
  n, d_in = x.shape
  d_h = W1.shape[1]
  n_classes = Wo.shape[1]
  e = edge_index.shape[1]
  n_pad = _round_up(n + 1, 128)
  e_pad = _round_up(e + n, NCS * NS * K_EDGES * 2)
  e_tot = e_pad + 2 * K_EDGES

  ei = edge_index.astype(jnp.int32)
  loop = jnp.arange(n, dtype=jnp.int32)
  pad_e = e_tot - e - n
  srcf = jnp.concatenate(
      [ei[0], loop, jnp.zeros((pad_e,), jnp.int32)]).reshape(-1, K_EDGES)
  dstf = jnp.concatenate(
      [ei[1], loop, jnp.full((pad_e,), n, jnp.int32)]).reshape(-1, K_EDGES)
  sd = jnp.stack([srcf, dstf], axis=1)

  x_pad = jnp.pad(x, ((0, n_pad - n), (0, 0)))

  vec = lambda a: a.reshape(1, -1)

  h1, asad1 = pl.pallas_call(
      _tc_in_body,
      out_shape=[jax.ShapeDtypeStruct((n_pad, d_h), jnp.float32),
                 jax.ShapeDtypeStruct((2, n_pad), jnp.float32)],
  )(x_pad, W1, vec(a_src1), vec(a_dst1))

  acc1, den1 = _sc_edge_call(n_pad, e_pad, d_h, h1, sd, asad1)

  h2, asad2 = pl.pallas_call(
      _tc_mid_body,
      out_shape=[jax.ShapeDtypeStruct((n_pad, d_h), jnp.float32),
                 jax.ShapeDtypeStruct((2, n_pad), jnp.float32)],
  )(acc1, den1, vec(b1), W2, vec(a_src2), vec(a_dst2))

  acc2, den2 = _sc_edge_call(n_pad, e_pad, d_h, h2, sd, asad2)

  out = pl.pallas_call(
      _tc_head_body,
      out_shape=jax.ShapeDtypeStruct((n_pad, n_classes), jnp.float32),
  )(acc2, den2, vec(b2), Wo, vec(bo))

  return out[:n]
